# baseline (speedup 1.0000x reference)
import functools

import jax
import jax.numpy as jnp
from jax import lax
from jax.experimental import pallas as pl
from jax.experimental.pallas import tpu as pltpu
from jax.experimental.pallas import tpu_sc as plsc

_N, _C, _B, _S = 10000, 256, 64, 32
_NSEG = _B * _S
_ROWS, _LANES = 80, 128
_NPAD = _ROWS * _LANES
_CAND = 100
_K_LO = 101

_SC_CORES, _SC_SUBCORES = 2, 16
_NW = _SC_CORES * _SC_SUBCORES
_GPW = _NSEG // _NW
_BLK = 2000


def _sc_gather(x, node_idx):
    mesh = plsc.VectorSubcoreMesh(core_axis_name="c", subcore_axis_name="s")

    @functools.partial(
        pl.kernel,
        mesh=mesh,
        out_type=jax.ShapeDtypeStruct((_NSEG, _C), jnp.float32),
        scratch_types=[
            pltpu.VMEM((_GPW,), jnp.int32),
            pltpu.VMEM((_GPW, _C), jnp.float32),
            pltpu.SemaphoreType.DMA,
        ],
    )
    def k(x_hbm, idx_hbm, out_hbm, idx_v, rows_v, sem):
        wid = lax.axis_index("s") * _SC_CORES + lax.axis_index("c")
        base = wid * _GPW
        pltpu.sync_copy(idx_hbm.at[pl.ds(base, _GPW)], idx_v)
        pltpu.async_copy(x_hbm.at[idx_v], rows_v, sem).wait()
        pltpu.sync_copy(rows_v, out_hbm.at[pl.ds(base, _GPW)])

    return k(x, node_idx)


def _proto_body(xg_ref, wl_ref, w1_ref, b1_ref, w2_ref, b2_ref,
                proto_ref, pn_ref):
    xg = xg_ref[...]
    h = lax.dot_general(xg, wl_ref[...], (((1,), (1,)), ((), ())),
                        preferred_element_type=jnp.float32)
    s1 = lax.dot_general(h, w1_ref[...], (((1,), (1,)), ((), ())),
                         preferred_element_type=jnp.float32)
    s1 = jnp.maximum(s1 + b1_ref[...][None, :], 0.0)
    w2p = jnp.where(
        lax.broadcasted_iota(jnp.int32, (8, _C // 2), 0) == 0,
        jnp.broadcast_to(w2_ref[...], (8, _C // 2)), 0.0)
    scm = lax.dot_general(s1, w2p, (((1,), (1,)), ((), ())),
                          preferred_element_type=jnp.float32)
    sc = scm[:, 0:1] + b2_ref[0]
    sc2 = sc.reshape(_B, _S)
    m = jnp.max(sc2, axis=1, keepdims=True)
    e = jnp.exp(sc2 - m)
    ssum = jnp.sum(e, axis=1, keepdims=True)
    att = (e / ssum).reshape(_NSEG, 1)
    w3 = (h * att).reshape(_B, _S, _C)
    proto = jnp.sum(w3, axis=1)
    proto_ref[...] = proto
    n = jnp.sqrt(jnp.sum(proto * proto, axis=1, keepdims=True))
    pn_ref[...] = proto / jnp.maximum(n, 1e-12)


def _proto_tc(xg, W_lin, att_W1, att_b1, att_W2, att_b2):
    return pl.pallas_call(
        _proto_body,
        in_specs=[
            pl.BlockSpec(memory_space=pltpu.VMEM),
            pl.BlockSpec(memory_space=pltpu.VMEM),
            pl.BlockSpec(memory_space=pltpu.VMEM),
            pl.BlockSpec(memory_space=pltpu.VMEM),
            pl.BlockSpec(memory_space=pltpu.VMEM),
            pl.BlockSpec(memory_space=pltpu.SMEM),
        ],
        out_shape=(jax.ShapeDtypeStruct((_B, _C), jnp.float32),
                   jax.ShapeDtypeStruct((_B, _C), jnp.float32)),
    )(xg, W_lin, att_W1, att_b1, att_W2, att_b2)


def _sim_body(x_ref, wl_ref, pn_ref, sm_ref, am_ref):
    xb = x_ref[...]
    xp = lax.dot_general(xb, wl_ref[...], (((1,), (1,)), ((), ())),
                         preferred_element_type=jnp.float32)
    n = jnp.sqrt(jnp.sum(xp * xp, axis=1, keepdims=True))
    xn = xp / jnp.maximum(n, 1e-12)
    t = lax.dot_general(xn, pn_ref[...], (((1,), (1,)), ((), ())),
                        preferred_element_type=jnp.float32)
    sm = jnp.max(t, axis=1, keepdims=True)
    col = lax.broadcasted_iota(jnp.int32, t.shape, 1)
    am = jnp.min(jnp.where(t == sm, col, _B), axis=1, keepdims=True)
    sm_ref[...] = sm[None]
    am_ref[...] = am[None]


def _sim_tc(x, W_lin, pn):
    nblk = _N // _BLK
    return pl.pallas_call(
        _sim_body,
        grid=(nblk,),
        in_specs=[
            pl.BlockSpec((_BLK, _C), lambda i: (i, 0)),
            pl.BlockSpec((_C, _C), lambda i: (0, 0)),
            pl.BlockSpec((_B, _C), lambda i: (0, 0)),
        ],
        out_specs=(
            pl.BlockSpec((1, _BLK, 1), lambda i: (i, 0, 0)),
            pl.BlockSpec((1, _BLK, 1), lambda i: (i, 0, 0)),
        ),
        out_shape=(jax.ShapeDtypeStruct((nblk, _BLK, 1), jnp.float32),
                   jax.ShapeDtypeStruct((nblk, _BLK, 1), jnp.int32)),
    )(x, W_lin, pn)


def _select_body(sm_ref, am_ref, nid_ref, fm_ref, ci_ref, ca_ref):
    sm = sm_ref[...]
    u = lax.bitcast_convert_type(sm, jnp.uint32)
    big = jnp.uint32(0x80000000)
    key = jnp.where(u >= big, ~u, u | big)

    def bstep(i, ans):
        bit = lax.shift_left(jnp.uint32(1), jnp.uint32(31) - i.astype(jnp.uint32))
        cand = ans | bit
        cnt = jnp.sum((key >= cand).astype(jnp.int32))
        return jnp.where(cnt >= _K_LO, cand, ans)

    keylo = lax.fori_loop(0, 32, bstep, jnp.uint32(0))
    ulo = jnp.where(keylo >= big, keylo ^ big, ~keylo)
    v_lo = lax.bitcast_convert_type(ulo, jnp.float32)

    c_ge = jnp.sum((sm >= v_lo).astype(jnp.int32))
    v_above = jnp.min(jnp.where(sm > v_lo, sm, jnp.inf))
    v_hi = jnp.where(c_ge > _K_LO, v_lo, v_above)

    qi = jnp.float32(0.99) * jnp.float32(_N - 1)
    hw = qi - jnp.floor(qi)
    lw = jnp.float32(1.0) - hw
    thresh = v_lo * lw + v_hi * hw

    cm = (sm > thresh)
    cmf = cm.astype(jnp.float32)
    total = jnp.sum(cm.astype(jnp.int32))

    tri_l = lax.broadcasted_iota(jnp.int32, (_LANES, _LANES), 0)
    tri_u = (tri_l <= lax.broadcasted_iota(jnp.int32, (_LANES, _LANES), 1))
    rowpre = lax.dot_general(cmf, tri_u.astype(jnp.float32),
                             (((1,), (0,)), ((), ())),
                             preferred_element_type=jnp.float32)
    row_tot = rowpre[:, _LANES - 1:_LANES]
    rtri = (lax.broadcasted_iota(jnp.int32, (_ROWS, _ROWS), 1)
            < lax.broadcasted_iota(jnp.int32, (_ROWS, _ROWS), 0))
    offs = lax.dot_general(rtri.astype(jnp.float32), row_tot,
                           (((1,), (0,)), ((), ())),
                           preferred_element_type=jnp.float32)
    cum = (rowpre + offs).astype(jnp.int32)

    tk = lax.broadcasted_iota(jnp.int32, (1, 1, _CAND), 2) + 1
    cmp = (cum[:, :, None] < tk).astype(jnp.int32)
    idx100 = jnp.sum(cmp, axis=(0, 1))
    kk = lax.broadcasted_iota(jnp.int32, (_CAND,), 0)
    ci = jnp.where(kk < total, idx100, 0)
    ci_ref[...] = ci

    pos = (lax.broadcasted_iota(jnp.int32, (_ROWS, _LANES), 0) * _LANES
           + lax.broadcasted_iota(jnp.int32, (_ROWS, _LANES), 1))
    match = (pos[:, :, None] == ci[None, None, :])
    am = am_ref[...]
    ca_ref[...] = jnp.sum(jnp.where(match, am[:, :, None], 0), axis=(0, 1))

    nid = nid_ref[...]
    m1 = ((nid >> 7) == lax.broadcasted_iota(jnp.int32, (1, _ROWS), 1)
          ).astype(jnp.float32)
    m2 = ((nid & 127) == lax.broadcasted_iota(jnp.int32, (1, _LANES), 1)
          ).astype(jnp.float32)
    hcnt = lax.dot_general(m1, m2, (((0,), (0,)), ((), ())),
                           preferred_element_type=jnp.float32)
    hf = (hcnt > 0.5).astype(jnp.float32)
    fm_ref[...] = hf + (1.0 - hf) * cmf


def _select_tc(smp, amp, nid):
    return pl.pallas_call(
        _select_body,
        out_shape=(jax.ShapeDtypeStruct((_ROWS, _LANES), jnp.float32),
                   jax.ShapeDtypeStruct((_CAND,), jnp.int32),
                   jax.ShapeDtypeStruct((_CAND,), jnp.int32)),
    )(smp, amp, nid)


def kernel(x, edge_index, edge_weight, subG_node, W_lin,
           att_W1, att_b1, att_W2, att_b2):
    del edge_index, edge_weight
    node_idx = subG_node.reshape(-1).astype(jnp.int32)
    xg = x[node_idx]
    proto, pn = _proto_tc(xg, W_lin, att_W1, att_b1, att_W2, att_b2)
    sm3, am3 = _sim_tc(x, W_lin, pn)
    sm = sm3.reshape(_N)
    am = am3.reshape(_N)
    smp = jnp.pad(sm, (0, _NPAD - _N),
                  constant_values=-jnp.inf).reshape(_ROWS, _LANES)
    amp = jnp.pad(am, (0, _NPAD - _N)).reshape(_ROWS, _LANES)
    fm, ci, ca = _select_tc(smp, amp, node_idx.reshape(_NSEG, 1))
    final_mask = fm.reshape(_NPAD)[:_N][:, None]
    return final_mask, proto, ci, ca


# speedup vs baseline: 1.1796x; 1.1796x over previous
import functools

import jax
import jax.numpy as jnp
from jax import lax
from jax.experimental import pallas as pl
from jax.experimental.pallas import tpu as pltpu
from jax.experimental.pallas import tpu_sc as plsc

_N, _C, _B, _S = 10000, 256, 64, 32
_NSEG = _B * _S
_CAND = 100
_K_LO = 101

_SC_CORES, _SC_SUBCORES = 2, 16
_NW = _SC_CORES * _SC_SUBCORES
_GPW = _NSEG // _NW
_BLK = 2000
_NBLK = _N // _BLK


def _sc_gather(x, node_idx):
    mesh = plsc.VectorSubcoreMesh(core_axis_name="c", subcore_axis_name="s")

    @functools.partial(
        pl.kernel,
        mesh=mesh,
        out_type=jax.ShapeDtypeStruct((_NSEG, _C), jnp.float32),
        scratch_types=[
            pltpu.VMEM((_GPW,), jnp.int32),
            pltpu.VMEM((_GPW, _C), jnp.float32),
            pltpu.SemaphoreType.DMA,
        ],
    )
    def k(x_hbm, idx_hbm, out_hbm, idx_v, rows_v, sem):
        wid = lax.axis_index("s") * _SC_CORES + lax.axis_index("c")
        base = wid * _GPW
        pltpu.sync_copy(idx_hbm.at[pl.ds(base, _GPW)], idx_v)
        pltpu.async_copy(x_hbm.at[idx_v], rows_v, sem).wait()
        pltpu.sync_copy(rows_v, out_hbm.at[pl.ds(base, _GPW)])

    return k(x, node_idx)


def _fused_body(x_ref, wl_ref, xg_ref, w1_ref, b1_ref, w2_ref, b2_ref,
                proto_ref, sm_ref, am_ref, pn_scr):
    pid = pl.program_id(0)

    @pl.when(pid == 0)
    def _proto():
        xg = xg_ref[...]
        h = lax.dot_general(xg, wl_ref[...], (((1,), (1,)), ((), ())),
                            preferred_element_type=jnp.float32)
        s1 = lax.dot_general(h, w1_ref[...], (((1,), (1,)), ((), ())),
                             preferred_element_type=jnp.float32)
        s1 = jnp.maximum(s1 + b1_ref[...][None, :], 0.0)
        w2p = jnp.where(
            lax.broadcasted_iota(jnp.int32, (8, _C // 2), 0) == 0,
            jnp.broadcast_to(w2_ref[...], (8, _C // 2)), 0.0)
        scm = lax.dot_general(s1, w2p, (((1,), (1,)), ((), ())),
                              preferred_element_type=jnp.float32)
        sc = scm[:, 0:1] + b2_ref[0]
        sc2 = sc.reshape(_B, _S)
        m = jnp.max(sc2, axis=1, keepdims=True)
        e = jnp.exp(sc2 - m)
        ssum = jnp.sum(e, axis=1, keepdims=True)
        att = (e / ssum).reshape(_NSEG, 1)
        w3 = (h * att).reshape(_B, _S, _C)
        proto = jnp.sum(w3, axis=1)
        proto_ref[...] = proto
        n = jnp.sqrt(jnp.sum(proto * proto, axis=1, keepdims=True))
        pn_scr[...] = proto / jnp.maximum(n, 1e-12)

    @pl.when(pid > 0)
    def _sim():
        xb = x_ref[...]
        xp = lax.dot_general(xb, wl_ref[...], (((1,), (1,)), ((), ())),
                             preferred_element_type=jnp.float32)
        n = jnp.sqrt(jnp.sum(xp * xp, axis=1, keepdims=True))
        xn = xp / jnp.maximum(n, 1e-12)
        tt = lax.dot_general(pn_scr[...], xn, (((1,), (1,)), ((), ())),
                             preferred_element_type=jnp.float32)
        smr = jnp.max(tt, axis=0, keepdims=True)
        rowi = lax.broadcasted_iota(jnp.int32, tt.shape, 0)
        amr = jnp.min(jnp.where(tt == smr, rowi, _B), axis=0, keepdims=True)
        sm_ref[...] = smr[None]
        am_ref[...] = amr[None]


def _fused_tc(x, W_lin, xg, att_W1, att_b1, att_W2, att_b2):
    return pl.pallas_call(
        _fused_body,
        grid=(_NBLK + 1,),
        in_specs=[
            pl.BlockSpec((_BLK, _C), lambda i: (jnp.maximum(i - 1, 0), 0)),
            pl.BlockSpec((_C, _C), lambda i: (0, 0)),
            pl.BlockSpec((_NSEG, _C), lambda i: (0, 0)),
            pl.BlockSpec((_C // 2, _C), lambda i: (0, 0)),
            pl.BlockSpec((_C // 2,), lambda i: (0,)),
            pl.BlockSpec((1, _C // 2), lambda i: (0, 0)),
            pl.BlockSpec(memory_space=pltpu.SMEM),
        ],
        out_specs=(
            pl.BlockSpec((_B, _C), lambda i: (0, 0)),
            pl.BlockSpec((1, 1, _BLK), lambda i: (jnp.maximum(i - 1, 0), 0, 0)),
            pl.BlockSpec((1, 1, _BLK), lambda i: (jnp.maximum(i - 1, 0), 0, 0)),
        ),
        out_shape=(jax.ShapeDtypeStruct((_B, _C), jnp.float32),
                   jax.ShapeDtypeStruct((_NBLK, 1, _BLK), jnp.float32),
                   jax.ShapeDtypeStruct((_NBLK, 1, _BLK), jnp.int32)),
        scratch_shapes=[pltpu.VMEM((_B, _C), jnp.float32)],
    )(x, W_lin, xg, att_W1, att_b1, att_W2, att_b2)


def _select_body(sm_ref, am_ref, nid_ref, fm_ref, ci_ref, ca_ref):
    sm = jnp.squeeze(sm_ref[...], axis=1)
    u = lax.bitcast_convert_type(sm, jnp.uint32)
    big = jnp.uint32(0x80000000)
    key = jnp.where(u >= big, ~u, u | big)

    def bstep(i, ans):
        bit = lax.shift_left(jnp.uint32(1), jnp.uint32(31) - i.astype(jnp.uint32))
        cand = ans | bit
        cnt = jnp.sum((key >= cand).astype(jnp.int32))
        return jnp.where(cnt >= _K_LO, cand, ans)

    keylo = lax.fori_loop(0, 32, bstep, jnp.uint32(0))
    ulo = jnp.where(keylo >= big, keylo ^ big, ~keylo)
    v_lo = lax.bitcast_convert_type(ulo, jnp.float32)

    c_ge = jnp.sum((sm >= v_lo).astype(jnp.int32))
    v_above = jnp.min(jnp.where(sm > v_lo, sm, jnp.inf))
    v_hi = jnp.where(c_ge > _K_LO, v_lo, v_above)

    qi = jnp.float32(0.99) * jnp.float32(_N - 1)
    hw = qi - jnp.floor(qi)
    lw = jnp.float32(1.0) - hw
    thresh = v_lo * lw + v_hi * hw

    cm = (sm > thresh)
    cmf = cm.astype(jnp.float32)
    total = jnp.sum(cm.astype(jnp.int32))

    s = cm.astype(jnp.int32)
    d = 1
    while d < _BLK:
        s = s + jnp.concatenate(
            [jnp.zeros((_NBLK, d), jnp.int32), s[:, :_BLK - d]], axis=1)
        d *= 2
    rowtot = s[:, _BLK - 1:_BLK]
    r = rowtot
    d = 1
    while d < _NBLK:
        r = r + jnp.concatenate(
            [jnp.zeros((d, 1), jnp.int32), r[:_NBLK - d, :]], axis=0)
        d *= 2
    cum = s + (r - rowtot)

    tk = lax.broadcasted_iota(jnp.int32, (1, 1, _CAND), 2) + 1
    cmp = (cum[:, :, None] < tk).astype(jnp.int32)
    idx100 = jnp.sum(cmp, axis=(0, 1))
    kk = lax.broadcasted_iota(jnp.int32, (_CAND,), 0)
    ci = jnp.where(kk < total, idx100, 0)
    ci_ref[...] = ci

    pos = (lax.broadcasted_iota(jnp.int32, (_NBLK, _BLK), 0) * _BLK
           + lax.broadcasted_iota(jnp.int32, (_NBLK, _BLK), 1))
    match = (pos[:, :, None] == ci[None, None, :])
    am = jnp.squeeze(am_ref[...], axis=1)
    ca_ref[...] = jnp.sum(jnp.where(match, am[:, :, None], 0), axis=(0, 1))

    nid = nid_ref[...]
    q = nid // _BLK
    m1 = (q == lax.broadcasted_iota(jnp.int32, (1, _NBLK), 1)
          ).astype(jnp.float32)
    m2 = ((nid - q * _BLK) == lax.broadcasted_iota(jnp.int32, (1, _BLK), 1)
          ).astype(jnp.float32)
    hcnt = lax.dot_general(m1, m2, (((0,), (0,)), ((), ())),
                           preferred_element_type=jnp.float32)
    hf = (hcnt > 0.5).astype(jnp.float32)
    fm_ref[...] = hf + (1.0 - hf) * cmf


def _select_tc(sm3, am3, nid):
    return pl.pallas_call(
        _select_body,
        out_shape=(jax.ShapeDtypeStruct((_NBLK, _BLK), jnp.float32),
                   jax.ShapeDtypeStruct((_CAND,), jnp.int32),
                   jax.ShapeDtypeStruct((_CAND,), jnp.int32)),
    )(sm3, am3, nid)


def kernel(x, edge_index, edge_weight, subG_node, W_lin,
           att_W1, att_b1, att_W2, att_b2):
    del edge_index, edge_weight
    node_idx = subG_node.reshape(-1).astype(jnp.int32)
    xg = _sc_gather(x, node_idx)
    proto, sm3, am3 = _fused_tc(x, W_lin, xg, att_W1, att_b1, att_W2, att_b2)
    fm, ci, ca = _select_tc(sm3, am3, node_idx.reshape(_NSEG, 1))
    final_mask = fm.reshape(_N, 1)
    return final_mask, proto, ci, ca
